# baseline (speedup 1.0000x reference)
import functools

import jax
import jax.numpy as jnp
from jax import lax
from jax.experimental import pallas as pl
from jax.experimental.pallas import tpu as pltpu
from jax.experimental.pallas import tpu_sc as plsc

_INFO = plsc.get_sparse_core_info()
_NC = _INFO.num_cores
_NS = _INFO.num_subcores
_L = _INFO.num_lanes
_NW = _NC * _NS

_CR = 8
_GR = 3


def kernel(x, pos_table):
    B, S, D = x.shape
    RW = S // _NW
    NCH = RW // _CR
    NSL = _CR * D // _L
    CSL = D // _L

    @functools.partial(
        pl.kernel,
        mesh=plsc.VectorSubcoreMesh(core_axis_name="c", subcore_axis_name="s"),
        out_type=jax.ShapeDtypeStruct((B, S, D), jnp.float32),
        scratch_types=(
            [pltpu.VMEM((_CR, D), jnp.float32) for _ in range(2)]
            + [pltpu.VMEM((_CR, D), jnp.float32) for _ in range(_GR * B)]
            + [
                pltpu.SemaphoreType.DMA((2,)),
                pltpu.SemaphoreType.DMA((_GR * B,)),
                pltpu.SemaphoreType.DMA((_GR * B,)),
            ]
        ),
        compiler_params=pltpu.CompilerParams(use_tc_tiling_on_sc=True),
    )
    def run(x_hbm, pos_hbm, out_hbm, p0, p1, *rest):
        xbufs = list(rest[: _GR * B])
        sp, sl, ss = rest[_GR * B:]
        pbufs = [p0, p1]
        wid = lax.axis_index("s") * _NC + lax.axis_index("c")
        base = wid * RW

        def rows(g):
            return pl.ds(base + g * _CR, _CR)

        def group_bufs(g):
            return xbufs[(g % _GR) * B:(g % _GR) * B + B]

        def issue_loads(g, load_d):
            for b in range(B):
                i = (g % _GR) * B + b
                load_d[(g, b)] = pltpu.async_copy(
                    x_hbm.at[b, rows(g)], xbufs[i], sl.at[i])

        pos_d, load_d, store_d = {}, {}, {}
        for c in range(min(2, NCH)):
            pos_d[c] = pltpu.async_copy(
                pos_hbm.at[rows(c)], pbufs[c], sp.at[c])
        for g in range(min(2, NCH)):
            issue_loads(g, load_d)

        for g in range(NCH):
            pos_d[g].wait()
            for b in range(B):
                load_d[(g, b)].wait()
            pb = pbufs[g % 2]
            gb = tuple(group_bufs(g))

            @plsc.parallel_loop(0, NSL, unroll=4)
            def add_body(i, pb=pb, gb=gb):
                r = i // CSL
                col = pl.ds((i % CSL) * _L, _L)
                v = pb[r, col]
                for xb in gb:
                    plsc.addupdate(xb.at[r, col], v)

            for b in range(B):
                i = (g % _GR) * B + b
                store_d[(g, b)] = pltpu.async_copy(
                    xbufs[i], out_hbm.at[b, rows(g)], ss.at[i])
            if g + 2 < NCH:
                if g - 1 >= 0:
                    for b in range(B):
                        store_d.pop((g - 1, b)).wait()
                issue_loads(g + 2, load_d)
                pos_d[g + 2] = pltpu.async_copy(
                    pos_hbm.at[rows(g + 2)], pbufs[g % 2], sp.at[g % 2])

        for key in sorted(store_d):
            store_d[key].wait()

    return run(x, pos_table[:S])
